# baseline (speedup 1.0000x reference)
import functools

import jax
import jax.numpy as jnp
import numpy as np
from jax import lax
from jax.experimental import pallas as pl
from jax.experimental.pallas import tpu as pltpu
from jax.experimental.pallas import tpu_sc as plsc

N = 10000
NPAD = 10240
E = 320000
D_IN = 128
H = 32
C = 2

NC = 2
NS = 16
NW = NC * NS
PER_W = E // NW
K = 80
NCHUNK = PER_W // K
CW = 16
RPT = NPAD // NS


def _segsum_kernel(with_counts):
    mesh = plsc.VectorSubcoreMesh(core_axis_name="c", subcore_axis_name="s")
    if with_counts:
        out_type = [jax.ShapeDtypeStruct((NC, NPAD, H), jnp.float32),
                    jax.ShapeDtypeStruct((NC, NPAD, CW), jnp.float32)]
    else:
        out_type = jax.ShapeDtypeStruct((NC, NPAD, H), jnp.float32)
    NBUF = 5
    scratch = (
        [pltpu.VMEM((PER_W,), jnp.int32),
         pltpu.VMEM((NCHUNK, K), jnp.int32),
         pltpu.VMEM((PER_W,), jnp.float32)]
        + [pltpu.VMEM((K, H), jnp.bfloat16)] * NBUF
        + [pltpu.VMEM((K, H), jnp.float32)] * NBUF
        + [pltpu.VMEM((RPT, H), jnp.float32),
           pltpu.VMEM_SHARED((NPAD, H), jnp.float32)]
        + [pltpu.SemaphoreType.DMA] * (2 * NBUF)
    )
    if with_counts:
        scratch += [
            pltpu.VMEM((K, CW), jnp.float32),
            pltpu.VMEM((RPT, CW), jnp.float32),
            pltpu.VMEM_SHARED((NPAD, CW), jnp.float32),
            pltpu.SemaphoreType.DMA,
        ]

    def body(p_hbm, src_hbm, dst_hbm, w_hbm, out_hbm, *rest):
        rest = list(rest)
        cnt_hbm = rest.pop(0) if with_counts else None
        srcb, dstb, wb = rest[0:3]
        bufs = rest[3:3 + NBUF]
        sbufs = rest[3 + NBUF:3 + 2 * NBUF]
        zbuf = rest[3 + 2 * NBUF]
        acc = rest[4 + 2 * NBUF]
        gsem = rest[5 + 2 * NBUF:5 + 3 * NBUF]
        ssem = rest[5 + 3 * NBUF:5 + 4 * NBUF]
        if with_counts:
            ones, zcnt, cacc, csem = rest[5 + 4 * NBUF:]
        c = lax.axis_index("c")
        s = lax.axis_index("s")
        wid = c * NS + s

        pltpu.async_copy(src_hbm.at[wid], srcb, gsem[0])
        pltpu.async_copy(dst_hbm.at[wid], dstb, gsem[0])
        pltpu.async_copy(w_hbm.at[wid], wb, gsem[0])

        @pl.loop(0, RPT)
        def _zero(i):
            for j in range(0, H, 16):
                zbuf[i, pl.ds(j, 16)] = jnp.zeros((16,), jnp.float32)
            if with_counts:
                zcnt[i, pl.ds(0, CW)] = jnp.zeros((CW,), jnp.float32)

        pltpu.sync_copy(zbuf, acc.at[pl.ds(s * RPT, RPT)])
        if with_counts:
            pltpu.sync_copy(zcnt, cacc.at[pl.ds(s * RPT, RPT)])

            @pl.loop(0, K)
            def _fill(i):
                ones[i, pl.ds(0, CW)] = jnp.ones((CW,), jnp.float32)

        pltpu.make_async_copy(src_hbm.at[wid], srcb, gsem[0]).wait()
        pltpu.make_async_copy(dst_hbm.at[wid], dstb, gsem[0]).wait()
        pltpu.make_async_copy(w_hbm.at[wid], wb, gsem[0]).wait()
        plsc.subcore_barrier()

        def start_gather(j, b):
            pltpu.async_copy(p_hbm.at[srcb.at[pl.ds(j * K, K)]],
                             bufs[b], gsem[b])

        def wait_gather(b):
            pltpu.make_async_copy(p_hbm.at[srcb.at[pl.ds(0, K)]], bufs[b],
                                  gsem[b]).wait()

        def wait_scatter(b):
            pltpu.make_async_copy(sbufs[b], acc.at[dstb.at[0]],
                                  ssem[b]).wait()

        def scale(b, j):
            buf = bufs[b]
            sbuf = sbufs[b]

            @pl.loop(0, K, step=16)
            def _scale(i):
                wv16 = wb[pl.ds(j * K + i, 16)]
                for e in range(16):
                    wv = wv16[e]
                    ab = buf[i + e, pl.ds(0, H)]
                    lo, hi = plsc.unpack(ab, format=plsc.PackFormat.INTERLEAVED)
                    sbuf[i + e, pl.ds(0, 16)] = lo * wv
                    sbuf[i + e, pl.ds(16, 16)] = hi * wv

        def step(j, b, first=False, last=False):
            wait_gather(b)
            scale(b, j)
            pltpu.async_copy(sbufs[b], acc.at[dstb.at[j]], ssem[b], add=True)
            if with_counts:
                if not first:
                    pltpu.make_async_copy(ones, cacc.at[dstb.at[0]],
                                          csem).wait()
                pltpu.async_copy(ones, cacc.at[dstb.at[j]], csem, add=True)
            if not last:
                bn = (b + 4) % NBUF
                if not first:
                    wait_scatter(bn)
                start_gather(j + 4, bn)

        for j in range(4):
            start_gather(j, j)
        step(0, 0, first=True)

        @pl.loop(0, 24)
        def _ring(t):
            j0 = 5 * t + 1
            for i in range(5):
                step(j0 + i, (1 + i) % NBUF)

        for j in range(NCHUNK - 4, NCHUNK):
            step(j, j % NBUF, last=True)

        for b in range(NBUF):
            wait_scatter(b)
        if with_counts:
            pltpu.make_async_copy(ones, cacc.at[dstb.at[0]], csem).wait()

        plsc.subcore_barrier()

        sl = pl.ds(s * RPT, RPT)
        pltpu.sync_copy(acc.at[sl], out_hbm.at[c, sl])
        if with_counts:
            pltpu.sync_copy(cacc.at[sl], cnt_hbm.at[c, sl])

    return pl.kernel(body, out_type=out_type, mesh=mesh,
                     scratch_types=scratch,
                     compiler_params=pltpu.CompilerParams(
                         use_tc_tiling_on_sc=False,
                         needs_layout_passes=False))


_segsum_counts = _segsum_kernel(True)
_segsum_plain = _segsum_kernel(False)


def _proj2_body(x_ref, wa_ref, wb_ref, pa_ref, pb_ref):
    xv = x_ref[...]
    pa_ref[...] = jnp.dot(
        xv, wa_ref[...],
        preferred_element_type=jnp.float32).astype(jnp.bfloat16)
    pb_ref[...] = jnp.dot(xv, wb_ref[...], preferred_element_type=jnp.float32)


def _proj2(x, wa, wb):
    m = x.shape[0]
    h = wa.shape[1]
    return pl.pallas_call(
        _proj2_body,
        out_shape=[jax.ShapeDtypeStruct((m, h), jnp.bfloat16),
                   jax.ShapeDtypeStruct((m, h), jnp.float32)],
    )(x, wa, wb)


def _mid_body(accp_ref, cntp_ref, r_ref, b_ref, wl_ref, wr_ref,
              p2_ref, r2_ref):
    sacc = accp_ref[0, :N] + accp_ref[1, :N]
    cnt = cntp_ref[0, :N, 0:1] + cntp_ref[1, :N, 0:1]
    aggr = sacc / jnp.maximum(cnt, 1.0)
    out = aggr + b_ref[...][None, :] + r_ref[...]
    nrm = jnp.sqrt(jnp.sum(out * out, axis=-1, keepdims=True))
    h = jax.nn.relu(out / jnp.maximum(nrm, 1e-12))
    p2_ref[...] = jnp.dot(
        h, wl_ref[...],
        preferred_element_type=jnp.float32).astype(jnp.bfloat16)
    r2_ref[...] = jnp.dot(h, wr_ref[...], preferred_element_type=jnp.float32)


def _final_body(accp_ref, cntp_ref, r_ref, b_ref, wlin_ref, blin_ref,
                o_ref):
    sacc = accp_ref[0, :N] + accp_ref[1, :N]
    cnt = cntp_ref[0, :N, 0:1] + cntp_ref[1, :N, 0:1]
    aggr = sacc / jnp.maximum(cnt, 1.0)
    out = aggr + b_ref[...][None, :] + r_ref[...]
    nrm = jnp.sqrt(jnp.sum(out * out, axis=-1, keepdims=True))
    h = jax.nn.relu(out / jnp.maximum(nrm, 1e-12))
    o_ref[...] = (jnp.dot(h, wlin_ref[...], preferred_element_type=jnp.float32)
                  + blin_ref[...][None, :])


def kernel(x, edge_index, edge_weight, W1l, b1l, W1r, W2l, b2l, W2r,
           Wlin, blin):
    src = edge_index[0].reshape(NW, PER_W)
    dst = edge_index[1].reshape(NW, NCHUNK, K)
    w2 = edge_weight.reshape(NW, PER_W)

    perm = np.concatenate([np.arange(0, H, 2), np.arange(1, H, 2)])

    p1, r1 = _proj2(x, W1l, W1r[:, perm])
    acc1, cnt = _segsum_counts(p1, src, dst, w2)

    p2, r2 = pl.pallas_call(
        _mid_body,
        out_shape=[jax.ShapeDtypeStruct((N, H), jnp.bfloat16),
                   jax.ShapeDtypeStruct((N, H), jnp.float32)],
    )(acc1, cnt, r1, b1l[perm], W2l[perm, :], W2r[perm][:, perm])

    acc2 = _segsum_plain(p2, src, dst, w2)

    out = pl.pallas_call(
        _final_body,
        out_shape=jax.ShapeDtypeStruct((N, C), jnp.float32),
    )(acc2, cnt, r2, b2l[perm], Wlin[perm, :], blin)
    return out


# speedup vs baseline: 1.2514x; 1.2514x over previous
import functools

import jax
import jax.numpy as jnp
import numpy as np
from jax import lax
from jax.experimental import pallas as pl
from jax.experimental.pallas import tpu as pltpu
from jax.experimental.pallas import tpu_sc as plsc

N = 10000
NPAD = 10240
E = 320000
D_IN = 128
H = 32
C = 2

NC = 2
NS = 16
NW = NC * NS
PER_W = E // NW
K = 80
NCHUNK = PER_W // K
CW = 16
RPT = NPAD // NS


def _segsum_kernel(with_counts):
    mesh = plsc.VectorSubcoreMesh(core_axis_name="c", subcore_axis_name="s")
    if with_counts:
        out_type = [jax.ShapeDtypeStruct((NC, NPAD, H), jnp.float32),
                    jax.ShapeDtypeStruct((NC, NPAD, CW), jnp.float32)]
    else:
        out_type = jax.ShapeDtypeStruct((NC, NPAD, H), jnp.float32)
    NBUF = 5
    scratch = (
        [pltpu.VMEM((PER_W,), jnp.int32),
         pltpu.VMEM((NCHUNK, K), jnp.int32),
         pltpu.VMEM((PER_W,), jnp.float32)]
        + [pltpu.VMEM((K, H), jnp.float32)] * NBUF
        + [pltpu.VMEM((RPT, H), jnp.float32),
           pltpu.VMEM_SHARED((NPAD, H), jnp.float32)]
        + [pltpu.SemaphoreType.DMA] * (2 * NBUF)
    )
    if with_counts:
        scratch += [
            pltpu.VMEM((K, CW), jnp.float32),
            pltpu.VMEM((RPT, CW), jnp.float32),
            pltpu.VMEM_SHARED((NPAD, CW), jnp.float32),
            pltpu.SemaphoreType.DMA,
        ]

    def body(p_hbm, src_hbm, dst_hbm, w_hbm, out_hbm, *rest):
        rest = list(rest)
        cnt_hbm = rest.pop(0) if with_counts else None
        srcb, dstb, wb = rest[0:3]
        bufs = rest[3:3 + NBUF]
        zbuf = rest[3 + NBUF]
        acc = rest[4 + NBUF]
        gsem = rest[5 + NBUF:5 + 2 * NBUF]
        ssem = rest[5 + 2 * NBUF:5 + 3 * NBUF]
        if with_counts:
            ones, zcnt, cacc, csem = rest[5 + 3 * NBUF:]
        c = lax.axis_index("c")
        s = lax.axis_index("s")
        wid = c * NS + s

        pltpu.async_copy(src_hbm.at[wid], srcb, gsem[0])
        pltpu.async_copy(dst_hbm.at[wid], dstb, gsem[0])
        pltpu.async_copy(w_hbm.at[wid], wb, gsem[0])

        @pl.loop(0, RPT)
        def _zero(i):
            for j in range(0, H, 16):
                zbuf[i, pl.ds(j, 16)] = jnp.zeros((16,), jnp.float32)
            if with_counts:
                zcnt[i, pl.ds(0, CW)] = jnp.zeros((CW,), jnp.float32)

        pltpu.sync_copy(zbuf, acc.at[pl.ds(s * RPT, RPT)])
        if with_counts:
            pltpu.sync_copy(zcnt, cacc.at[pl.ds(s * RPT, RPT)])

            @pl.loop(0, K)
            def _fill(i):
                ones[i, pl.ds(0, CW)] = jnp.ones((CW,), jnp.float32)

        pltpu.make_async_copy(src_hbm.at[wid], srcb, gsem[0]).wait()
        pltpu.make_async_copy(dst_hbm.at[wid], dstb, gsem[0]).wait()
        pltpu.make_async_copy(w_hbm.at[wid], wb, gsem[0]).wait()
        plsc.subcore_barrier()

        def start_gather(j, b):
            pltpu.async_copy(p_hbm.at[srcb.at[pl.ds(j * K, K)]],
                             bufs[b], gsem[b])

        def wait_gather(b):
            pltpu.make_async_copy(p_hbm.at[srcb.at[pl.ds(0, K)]], bufs[b],
                                  gsem[b]).wait()

        def wait_scatter(b):
            pltpu.make_async_copy(bufs[b], acc.at[dstb.at[0]],
                                  ssem[b]).wait()

        def scale(b, j):
            buf = bufs[b]

            @pl.loop(0, K, step=16)
            def _scale(i):
                wv16 = wb[pl.ds(j * K + i, 16)]
                for e in range(16):
                    wv = wv16[e]
                    for jj in range(0, H, 16):
                        buf[i + e, pl.ds(jj, 16)] = (
                            buf[i + e, pl.ds(jj, 16)] * wv)

        def step(j, b, first=False, last=False):
            wait_gather(b)
            scale(b, j)
            pltpu.async_copy(bufs[b], acc.at[dstb.at[j]], ssem[b], add=True)
            if with_counts:
                if not first:
                    pltpu.make_async_copy(ones, cacc.at[dstb.at[0]],
                                          csem).wait()
                pltpu.async_copy(ones, cacc.at[dstb.at[j]], csem, add=True)
            if not last:
                bn = (b + 4) % NBUF
                if not first:
                    wait_scatter(bn)
                start_gather(j + 4, bn)

        for j in range(4):
            start_gather(j, j)
        step(0, 0, first=True)

        @pl.loop(0, 24)
        def _ring(t):
            j0 = 5 * t + 1
            for i in range(5):
                step(j0 + i, (1 + i) % NBUF)

        for j in range(NCHUNK - 4, NCHUNK):
            step(j, j % NBUF, last=True)

        for b in range(NBUF):
            wait_scatter(b)
        if with_counts:
            pltpu.make_async_copy(ones, cacc.at[dstb.at[0]], csem).wait()

        plsc.subcore_barrier()

        sl = pl.ds(s * RPT, RPT)
        pltpu.sync_copy(acc.at[sl], out_hbm.at[c, sl])
        if with_counts:
            pltpu.sync_copy(cacc.at[sl], cnt_hbm.at[c, sl])

    return pl.kernel(body, out_type=out_type, mesh=mesh,
                     scratch_types=scratch,
                     compiler_params=pltpu.CompilerParams(
                         use_tc_tiling_on_sc=False))


_segsum_counts = _segsum_kernel(True)
_segsum_plain = _segsum_kernel(False)


def _proj2_body(x_ref, wa_ref, wb_ref, pa_ref, pb_ref):
    xv = x_ref[...]
    pa_ref[...] = jnp.dot(xv, wa_ref[...], preferred_element_type=jnp.float32)
    pb_ref[...] = jnp.dot(xv, wb_ref[...], preferred_element_type=jnp.float32)


NBLK = 10
BLK = N // NBLK


def _proj2(x, wa, wb):
    m = x.shape[0]
    d = x.shape[1]
    h = wa.shape[1]
    return pl.pallas_call(
        _proj2_body,
        grid=(NBLK,),
        in_specs=[pl.BlockSpec((BLK, d), lambda i: (i, 0)),
                  pl.BlockSpec((d, h), lambda i: (0, 0)),
                  pl.BlockSpec((d, h), lambda i: (0, 0))],
        out_specs=[pl.BlockSpec((BLK, h), lambda i: (i, 0))] * 2,
        out_shape=[jax.ShapeDtypeStruct((m, h), jnp.float32)] * 2,
    )(x, wa, wb)


def _mid_body(accp_ref, cntp_ref, r_ref, b_ref, wl_ref, wr_ref,
              p2_ref, r2_ref):
    sacc = accp_ref[0] + accp_ref[1]
    cnt = cntp_ref[0, :, 0:1] + cntp_ref[1, :, 0:1]
    aggr = sacc / jnp.maximum(cnt, 1.0)
    out = aggr + b_ref[...][None, :] + r_ref[...]
    nrm = jnp.sqrt(jnp.sum(out * out, axis=-1, keepdims=True))
    h = jax.nn.relu(out / jnp.maximum(nrm, 1e-12))
    p2_ref[...] = jnp.dot(h, wl_ref[...], preferred_element_type=jnp.float32)
    r2_ref[...] = jnp.dot(h, wr_ref[...], preferred_element_type=jnp.float32)


def _final_body(accp_ref, cntp_ref, r_ref, b_ref, wlin_ref, blin_ref,
                o_ref):
    sacc = accp_ref[0] + accp_ref[1]
    cnt = cntp_ref[0, :, 0:1] + cntp_ref[1, :, 0:1]
    aggr = sacc / jnp.maximum(cnt, 1.0)
    out = aggr + b_ref[...][None, :] + r_ref[...]
    nrm = jnp.sqrt(jnp.sum(out * out, axis=-1, keepdims=True))
    h = jax.nn.relu(out / jnp.maximum(nrm, 1e-12))
    o_ref[...] = (jnp.dot(h, wlin_ref[...], preferred_element_type=jnp.float32)
                  + blin_ref[...][None, :])


def kernel(x, edge_index, edge_weight, W1l, b1l, W1r, W2l, b2l, W2r,
           Wlin, blin):
    src = edge_index[0].reshape(NW, PER_W)
    dst = edge_index[1].reshape(NW, NCHUNK, K)
    w2 = edge_weight.reshape(NW, PER_W)

    p1, r1 = _proj2(x, W1l, W1r)
    acc1, cnt = _segsum_counts(p1, src, dst, w2)

    p2, r2 = pl.pallas_call(
        _mid_body,
        grid=(NBLK,),
        in_specs=[pl.BlockSpec((NC, BLK, H), lambda i: (0, i, 0)),
                  pl.BlockSpec((NC, BLK, CW), lambda i: (0, i, 0)),
                  pl.BlockSpec((BLK, H), lambda i: (i, 0)),
                  pl.BlockSpec((H,), lambda i: (0,)),
                  pl.BlockSpec((H, H), lambda i: (0, 0)),
                  pl.BlockSpec((H, H), lambda i: (0, 0))],
        out_specs=[pl.BlockSpec((BLK, H), lambda i: (i, 0))] * 2,
        out_shape=[jax.ShapeDtypeStruct((N, H), jnp.float32)] * 2,
    )(acc1, cnt, r1, b1l, W2l, W2r)

    acc2 = _segsum_plain(p2, src, dst, w2)

    out = pl.pallas_call(
        _final_body,
        grid=(NBLK,),
        in_specs=[pl.BlockSpec((NC, BLK, H), lambda i: (0, i, 0)),
                  pl.BlockSpec((NC, BLK, CW), lambda i: (0, i, 0)),
                  pl.BlockSpec((BLK, H), lambda i: (i, 0)),
                  pl.BlockSpec((H,), lambda i: (0,)),
                  pl.BlockSpec((H, C), lambda i: (0, 0)),
                  pl.BlockSpec((C,), lambda i: (0,))],
        out_specs=pl.BlockSpec((BLK, C), lambda i: (i, 0)),
        out_shape=jax.ShapeDtypeStruct((N, C), jnp.float32),
    )(acc2, cnt, r2, b2l, Wlin, blin)
    return out


# speedup vs baseline: 1.3212x; 1.0558x over previous
import functools

import jax
import jax.numpy as jnp
import numpy as np
from jax import lax
from jax.experimental import pallas as pl
from jax.experimental.pallas import tpu as pltpu
from jax.experimental.pallas import tpu_sc as plsc

N = 10000
NPAD = 10240
E = 320000
D_IN = 128
H = 32
C = 2

NC = 2
NS = 16
NW = NC * NS
PER_W = E // NW
K = 80
NCHUNK = PER_W // K
CW = 16
RPT = NPAD // NS


def _segsum_kernel(with_counts):
    mesh = plsc.VectorSubcoreMesh(core_axis_name="c", subcore_axis_name="s")
    if with_counts:
        out_type = [jax.ShapeDtypeStruct((NC, NPAD, H), jnp.float32),
                    jax.ShapeDtypeStruct((NC, NPAD, CW), jnp.float32)]
    else:
        out_type = jax.ShapeDtypeStruct((NC, NPAD, H), jnp.float32)
    NBUF = 5
    scratch = (
        [pltpu.VMEM((PER_W,), jnp.int32),
         pltpu.VMEM((NCHUNK, K), jnp.int32),
         pltpu.VMEM((PER_W,), jnp.float32)]
        + [pltpu.VMEM((K, H), jnp.float32)] * NBUF
        + [pltpu.VMEM((RPT, H), jnp.float32),
           pltpu.VMEM_SHARED((NPAD, H), jnp.float32)]
        + [pltpu.SemaphoreType.DMA] * (2 * NBUF)
    )
    if with_counts:
        scratch += [
            pltpu.VMEM((K, CW), jnp.float32),
            pltpu.VMEM((RPT, CW), jnp.float32),
            pltpu.VMEM_SHARED((NPAD, CW), jnp.float32),
            pltpu.SemaphoreType.DMA,
        ]

    def body(p_hbm, src_hbm, dst_hbm, w_hbm, out_hbm, *rest):
        rest = list(rest)
        cnt_hbm = rest.pop(0) if with_counts else None
        srcb, dstb, wb = rest[0:3]
        bufs = rest[3:3 + NBUF]
        zbuf = rest[3 + NBUF]
        acc = rest[4 + NBUF]
        gsem = rest[5 + NBUF:5 + 2 * NBUF]
        ssem = rest[5 + 2 * NBUF:5 + 3 * NBUF]
        if with_counts:
            ones, zcnt, cacc, csem = rest[5 + 3 * NBUF:]
        c = lax.axis_index("c")
        s = lax.axis_index("s")
        wid = c * NS + s

        pltpu.async_copy(src_hbm.at[wid], srcb, gsem[0])
        pltpu.async_copy(dst_hbm.at[wid], dstb, gsem[0])
        pltpu.async_copy(w_hbm.at[wid], wb, gsem[0])

        @pl.loop(0, RPT)
        def _zero(i):
            for j in range(0, H, 16):
                zbuf[i, pl.ds(j, 16)] = jnp.zeros((16,), jnp.float32)
            if with_counts:
                zcnt[i, pl.ds(0, CW)] = jnp.zeros((CW,), jnp.float32)

        pltpu.sync_copy(zbuf, acc.at[pl.ds(s * RPT, RPT)])
        if with_counts:
            pltpu.sync_copy(zcnt, cacc.at[pl.ds(s * RPT, RPT)])

            @pl.loop(0, K)
            def _fill(i):
                ones[i, pl.ds(0, CW)] = jnp.ones((CW,), jnp.float32)

        pltpu.make_async_copy(src_hbm.at[wid], srcb, gsem[0]).wait()
        pltpu.make_async_copy(dst_hbm.at[wid], dstb, gsem[0]).wait()
        pltpu.make_async_copy(w_hbm.at[wid], wb, gsem[0]).wait()
        plsc.subcore_barrier()

        def start_gather(j, b):
            pltpu.async_copy(p_hbm.at[srcb.at[pl.ds(j * K, K)]],
                             bufs[b], gsem[b])

        def wait_gather(b):
            pltpu.make_async_copy(p_hbm.at[srcb.at[pl.ds(0, K)]], bufs[b],
                                  gsem[b]).wait()

        def wait_scatter(b):
            pltpu.make_async_copy(bufs[b], acc.at[dstb.at[0]],
                                  ssem[b]).wait()

        def scale(b, j):
            buf = bufs[b]

            @pl.loop(0, K, step=16)
            def _scale(i):
                wv16 = wb[pl.ds(j * K + i, 16)]
                for e in range(16):
                    wv = wv16[e]
                    for jj in range(0, H, 16):
                        buf[i + e, pl.ds(jj, 16)] = (
                            buf[i + e, pl.ds(jj, 16)] * wv)

        def step(j, b, first=False, last=False):
            wait_gather(b)
            scale(b, j)
            pltpu.async_copy(bufs[b], acc.at[dstb.at[j]], ssem[b], add=True)
            if with_counts:
                if not first:
                    pltpu.make_async_copy(ones, cacc.at[dstb.at[0]],
                                          csem).wait()
                pltpu.async_copy(ones, cacc.at[dstb.at[j]], csem, add=True)
            if not last:
                bn = (b + 4) % NBUF
                if not first:
                    wait_scatter(bn)
                start_gather(j + 4, bn)

        for j in range(4):
            start_gather(j, j)
        step(0, 0, first=True)

        @pl.loop(0, 24)
        def _ring(t):
            j0 = 5 * t + 1
            for i in range(5):
                step(j0 + i, (1 + i) % NBUF)

        for j in range(NCHUNK - 4, NCHUNK):
            step(j, j % NBUF, last=True)

        for b in range(NBUF):
            wait_scatter(b)
        if with_counts:
            pltpu.make_async_copy(ones, cacc.at[dstb.at[0]], csem).wait()

        plsc.subcore_barrier()

        sl = pl.ds(s * RPT, RPT)
        pltpu.sync_copy(acc.at[sl], out_hbm.at[c, sl])
        if with_counts:
            pltpu.sync_copy(cacc.at[sl], cnt_hbm.at[c, sl])

    return pl.kernel(body, out_type=out_type, mesh=mesh,
                     scratch_types=scratch,
                     compiler_params=pltpu.CompilerParams(
                         use_tc_tiling_on_sc=False))


_segsum_counts = _segsum_kernel(True)
_segsum_plain = _segsum_kernel(False)


def _proj2_body(x_ref, wa_ref, wb_ref, pa_ref, pb_ref):
    xv = x_ref[...]
    pa_ref[...] = jnp.dot(xv, wa_ref[...], preferred_element_type=jnp.float32)
    pb_ref[...] = jnp.dot(xv, wb_ref[...], preferred_element_type=jnp.float32)


NBLK = 5
BLK = N // NBLK


def _proj2(x, wa, wb):
    m = x.shape[0]
    d = x.shape[1]
    h = wa.shape[1]
    return pl.pallas_call(
        _proj2_body,
        grid=(NBLK,),
        in_specs=[pl.BlockSpec((BLK, d), lambda i: (i, 0)),
                  pl.BlockSpec((d, h), lambda i: (0, 0)),
                  pl.BlockSpec((d, h), lambda i: (0, 0))],
        out_specs=[pl.BlockSpec((BLK, h), lambda i: (i, 0))] * 2,
        out_shape=[jax.ShapeDtypeStruct((m, h), jnp.float32)] * 2,
    )(x, wa, wb)


def _mid_body(accp_ref, cntp_ref, r_ref, b_ref, wl_ref, wr_ref,
              p2_ref, r2_ref):
    sacc = accp_ref[0] + accp_ref[1]
    cnt = cntp_ref[0, :, 0:1] + cntp_ref[1, :, 0:1]
    aggr = sacc / jnp.maximum(cnt, 1.0)
    out = aggr + b_ref[...][None, :] + r_ref[...]
    nrm = jnp.sqrt(jnp.sum(out * out, axis=-1, keepdims=True))
    h = jax.nn.relu(out / jnp.maximum(nrm, 1e-12))
    p2_ref[...] = jnp.dot(h, wl_ref[...], preferred_element_type=jnp.float32)
    r2_ref[...] = jnp.dot(h, wr_ref[...], preferred_element_type=jnp.float32)


def _final_body(accp_ref, cntp_ref, r_ref, b_ref, wlin_ref, blin_ref,
                o_ref):
    sacc = accp_ref[0] + accp_ref[1]
    cnt = cntp_ref[0, :, 0:1] + cntp_ref[1, :, 0:1]
    aggr = sacc / jnp.maximum(cnt, 1.0)
    out = aggr + b_ref[...][None, :] + r_ref[...]
    nrm = jnp.sqrt(jnp.sum(out * out, axis=-1, keepdims=True))
    h = jax.nn.relu(out / jnp.maximum(nrm, 1e-12))
    o_ref[...] = (jnp.dot(h, wlin_ref[...], preferred_element_type=jnp.float32)
                  + blin_ref[...][None, :])


def kernel(x, edge_index, edge_weight, W1l, b1l, W1r, W2l, b2l, W2r,
           Wlin, blin):
    src = edge_index[0].reshape(NW, PER_W)
    dst = edge_index[1].reshape(NW, NCHUNK, K)
    w2 = edge_weight.reshape(NW, PER_W)

    p1, r1 = _proj2(x, W1l, W1r)
    acc1, cnt = _segsum_counts(p1, src, dst, w2)

    p2, r2 = pl.pallas_call(
        _mid_body,
        grid=(NBLK,),
        in_specs=[pl.BlockSpec((NC, BLK, H), lambda i: (0, i, 0)),
                  pl.BlockSpec((NC, BLK, CW), lambda i: (0, i, 0)),
                  pl.BlockSpec((BLK, H), lambda i: (i, 0)),
                  pl.BlockSpec((H,), lambda i: (0,)),
                  pl.BlockSpec((H, H), lambda i: (0, 0)),
                  pl.BlockSpec((H, H), lambda i: (0, 0))],
        out_specs=[pl.BlockSpec((BLK, H), lambda i: (i, 0))] * 2,
        out_shape=[jax.ShapeDtypeStruct((N, H), jnp.float32)] * 2,
    )(acc1, cnt, r1, b1l, W2l, W2r)

    acc2 = _segsum_plain(p2, src, dst, w2)

    out = pl.pallas_call(
        _final_body,
        grid=(NBLK,),
        in_specs=[pl.BlockSpec((NC, BLK, H), lambda i: (0, i, 0)),
                  pl.BlockSpec((NC, BLK, CW), lambda i: (0, i, 0)),
                  pl.BlockSpec((BLK, H), lambda i: (i, 0)),
                  pl.BlockSpec((H,), lambda i: (0,)),
                  pl.BlockSpec((H, C), lambda i: (0, 0)),
                  pl.BlockSpec((C,), lambda i: (0,))],
        out_specs=pl.BlockSpec((BLK, C), lambda i: (i, 0)),
        out_shape=jax.ShapeDtypeStruct((N, C), jnp.float32),
    )(acc2, cnt, r2, b2l, Wlin, blin)
    return out


# speedup vs baseline: 1.5450x; 1.1693x over previous
import functools

import jax
import jax.numpy as jnp
import numpy as np
from jax import lax
from jax.experimental import pallas as pl
from jax.experimental.pallas import tpu as pltpu
from jax.experimental.pallas import tpu_sc as plsc

N = 10000
NPAD = 10240
E = 320000
D_IN = 128
H = 32
C = 2

NC = 2
NS = 16
NW = NC * NS
PER_W = E // NW
K = 80
NCHUNK = PER_W // K
CW = 32
RPT = NPAD // NS


def _segsum_kernel(with_counts):
    mesh = plsc.VectorSubcoreMesh(core_axis_name="c", subcore_axis_name="s")
    if with_counts:
        out_type = [jax.ShapeDtypeStruct((NC, NPAD, H), jnp.float32),
                    jax.ShapeDtypeStruct((NC, NPAD, CW), jnp.float32)]
    else:
        out_type = jax.ShapeDtypeStruct((NC, NPAD, H), jnp.float32)
    NBUF = 5
    scratch = (
        [pltpu.VMEM((PER_W,), jnp.int32),
         pltpu.VMEM((NCHUNK, K), jnp.int32),
         pltpu.VMEM((PER_W,), jnp.float32)]
        + [pltpu.VMEM((K, H), jnp.float32)] * NBUF
        + [pltpu.VMEM((RPT, H), jnp.float32),
           pltpu.VMEM_SHARED((NPAD, H), jnp.float32)]
        + [pltpu.SemaphoreType.DMA] * (2 * NBUF)
    )
    if with_counts:
        scratch += [
            pltpu.VMEM((K, CW), jnp.float32),
            pltpu.VMEM((RPT, CW), jnp.float32),
            pltpu.VMEM_SHARED((NPAD, CW), jnp.float32),
            pltpu.SemaphoreType.DMA,
        ]

    def body(p_hbm, src_hbm, dst_hbm, w_hbm, out_hbm, *rest):
        rest = list(rest)
        cnt_hbm = rest.pop(0) if with_counts else None
        srcb, dstb, wb = rest[0:3]
        bufs = rest[3:3 + NBUF]
        zbuf = rest[3 + NBUF]
        acc = rest[4 + NBUF]
        gsem = rest[5 + NBUF:5 + 2 * NBUF]
        ssem = rest[5 + 2 * NBUF:5 + 3 * NBUF]
        if with_counts:
            ones, zcnt, cacc, csem = rest[5 + 3 * NBUF:]
        c = lax.axis_index("c")
        s = lax.axis_index("s")
        wid = c * NS + s

        pltpu.async_copy(src_hbm.at[wid], srcb, gsem[0])
        pltpu.async_copy(dst_hbm.at[wid], dstb, gsem[0])
        pltpu.async_copy(w_hbm.at[wid], wb, gsem[0])

        @pl.loop(0, RPT)
        def _zero(i):
            for j in range(0, H, 16):
                zbuf[i, pl.ds(j, 16)] = jnp.zeros((16,), jnp.float32)
            if with_counts:
                for jj in range(0, CW, 16):
                    zcnt[i, pl.ds(jj, 16)] = jnp.zeros((16,), jnp.float32)

        pltpu.sync_copy(zbuf, acc.at[pl.ds(s * RPT, RPT)])
        if with_counts:
            pltpu.sync_copy(zcnt, cacc.at[pl.ds(s * RPT, RPT)])

            @pl.loop(0, K)
            def _fill(i):
                for jj in range(0, CW, 16):
                    ones[i, pl.ds(jj, 16)] = jnp.ones((16,), jnp.float32)

        pltpu.make_async_copy(src_hbm.at[wid], srcb, gsem[0]).wait()
        pltpu.make_async_copy(dst_hbm.at[wid], dstb, gsem[0]).wait()
        pltpu.make_async_copy(w_hbm.at[wid], wb, gsem[0]).wait()
        plsc.subcore_barrier()

        def start_gather(j, b):
            pltpu.async_copy(p_hbm.at[srcb.at[pl.ds(j * K, K)]],
                             bufs[b], gsem[b])

        def wait_gather(b):
            pltpu.make_async_copy(p_hbm.at[srcb.at[pl.ds(0, K)]], bufs[b],
                                  gsem[b]).wait()

        def wait_scatter(b):
            pltpu.make_async_copy(bufs[b], acc.at[dstb.at[0]],
                                  ssem[b]).wait()

        def scale(b, j):
            buf = bufs[b]

            @pl.loop(0, K, step=16)
            def _scale(i):
                wv16 = wb[pl.ds(j * K + i, 16)]
                for e in range(16):
                    wv = wv16[e]
                    for jj in range(0, H, 16):
                        buf[i + e, pl.ds(jj, 16)] = (
                            buf[i + e, pl.ds(jj, 16)] * wv)

        def step(j, b, first=False, last=False):
            wait_gather(b)
            scale(b, j)
            pltpu.async_copy(bufs[b], acc.at[dstb.at[j]], ssem[b], add=True)
            if with_counts:
                if not first:
                    pltpu.make_async_copy(ones, cacc.at[dstb.at[0]],
                                          csem).wait()
                pltpu.async_copy(ones, cacc.at[dstb.at[j]], csem, add=True)
            if not last:
                bn = (b + 4) % NBUF
                if not first:
                    wait_scatter(bn)
                start_gather(j + 4, bn)

        for j in range(4):
            start_gather(j, j)
        step(0, 0, first=True)

        @pl.loop(0, 24)
        def _ring(t):
            j0 = 5 * t + 1
            for i in range(5):
                step(j0 + i, (1 + i) % NBUF)

        for j in range(NCHUNK - 4, NCHUNK):
            step(j, j % NBUF, last=True)

        for b in range(NBUF):
            wait_scatter(b)
        if with_counts:
            pltpu.make_async_copy(ones, cacc.at[dstb.at[0]], csem).wait()

        plsc.subcore_barrier()

        sl = pl.ds(s * RPT, RPT)
        pltpu.sync_copy(acc.at[sl], out_hbm.at[c, sl])
        if with_counts:
            pltpu.sync_copy(cacc.at[sl], cnt_hbm.at[c, sl])

    return pl.kernel(body, out_type=out_type, mesh=mesh,
                     scratch_types=scratch,
                     compiler_params=pltpu.CompilerParams(
                         use_tc_tiling_on_sc=False))


_segsum_counts = _segsum_kernel(True)
_segsum_plain = _segsum_kernel(False)


PK = 4
NPK = NPAD // PK
NBLK = 5
BLK = NPK // NBLK


def _proj2_body(x_ref, wa_ref, wb_ref, pa_ref, pb_ref):
    xv = x_ref[...]
    pa_ref[...] = jnp.dot(xv, wa_ref[...], preferred_element_type=jnp.float32)
    pb_ref[...] = jnp.dot(xv, wb_ref[...], preferred_element_type=jnp.float32)


def _proj2(xw, wa_bd, wb_bd):
    return pl.pallas_call(
        _proj2_body,
        grid=(NBLK,),
        in_specs=[pl.BlockSpec((BLK, PK * D_IN), lambda i: (i, 0)),
                  pl.BlockSpec((PK * D_IN, 128), lambda i: (0, 0)),
                  pl.BlockSpec((PK * D_IN, 128), lambda i: (0, 0))],
        out_specs=[pl.BlockSpec((BLK, 128), lambda i: (i, 0))] * 2,
        out_shape=[jax.ShapeDtypeStruct((NPK, 128), jnp.float32)] * 2,
    )(xw, wa_bd, wb_bd)


def _norm_relu(aggr, b_p, r_p, bd1):
    out = aggr + b_p[None, :] + r_p
    nsq = jnp.dot(out * out, bd1, preferred_element_type=jnp.float32)
    nrm = jnp.sqrt(nsq)
    return jax.nn.relu(out / jnp.maximum(nrm, 1e-12))


def _mid_body(accp_ref, cntp_ref, r_ref, b_ref, wl_ref, wr_ref, bd1_ref,
              p2_ref, r2_ref):
    sacc = accp_ref[0] + accp_ref[1]
    cnt = cntp_ref[0] + cntp_ref[1]
    aggr = sacc / jnp.maximum(cnt, 1.0)
    h = _norm_relu(aggr, b_ref[...], r_ref[...], bd1_ref[...])
    p2_ref[...] = jnp.dot(h, wl_ref[...], preferred_element_type=jnp.float32)
    r2_ref[...] = jnp.dot(h, wr_ref[...], preferred_element_type=jnp.float32)


def _final_body(accp_ref, cntp_ref, r_ref, b_ref, wlin_ref, blin_ref,
                bd1_ref, o_ref):
    sacc = accp_ref[0] + accp_ref[1]
    cnt = cntp_ref[0] + cntp_ref[1]
    aggr = sacc / jnp.maximum(cnt, 1.0)
    h = _norm_relu(aggr, b_ref[...], r_ref[...], bd1_ref[...])
    o_ref[...] = (jnp.dot(h, wlin_ref[...], preferred_element_type=jnp.float32)
                  + blin_ref[...][None, :])


def kernel(x, edge_index, edge_weight, W1l, b1l, W1r, W2l, b2l, W2r,
           Wlin, blin):
    from jax.scipy.linalg import block_diag

    src = edge_index[0].reshape(NW, PER_W)
    dst = edge_index[1].reshape(NW, NCHUNK, K)
    w2 = edge_weight.reshape(NW, PER_W)

    bd = lambda w: block_diag(*([w] * PK))
    bd1 = bd(jnp.ones((H, H), jnp.float32))

    xw = jnp.pad(x.reshape(N // PK, PK * D_IN), ((0, NPK - N // PK), (0, 0)))
    p1p, r1p = _proj2(xw, bd(W1l), bd(W1r))
    acc1, cnt = _segsum_counts(p1p.reshape(NPAD, H), src, dst, w2)
    acc1v = acc1.reshape(NC, NPK, 128)
    cntv = cnt.reshape(NC, NPK, 128)

    p2p, r2p = pl.pallas_call(
        _mid_body,
        grid=(NBLK,),
        in_specs=[pl.BlockSpec((NC, BLK, 128), lambda i: (0, i, 0)),
                  pl.BlockSpec((NC, BLK, 128), lambda i: (0, i, 0)),
                  pl.BlockSpec((BLK, 128), lambda i: (i, 0)),
                  pl.BlockSpec((128,), lambda i: (0,)),
                  pl.BlockSpec((128, 128), lambda i: (0, 0)),
                  pl.BlockSpec((128, 128), lambda i: (0, 0)),
                  pl.BlockSpec((128, 128), lambda i: (0, 0))],
        out_specs=[pl.BlockSpec((BLK, 128), lambda i: (i, 0))] * 2,
        out_shape=[jax.ShapeDtypeStruct((NPK, 128), jnp.float32)] * 2,
    )(acc1v, cntv, r1p, jnp.tile(b1l, PK), bd(W2l), bd(W2r), bd1)

    acc2 = _segsum_plain(p2p.reshape(NPAD, H), src, dst, w2)
    acc2v = acc2.reshape(NC, NPK, 128)

    out = pl.pallas_call(
        _final_body,
        grid=(NBLK,),
        in_specs=[pl.BlockSpec((NC, BLK, 128), lambda i: (0, i, 0)),
                  pl.BlockSpec((NC, BLK, 128), lambda i: (0, i, 0)),
                  pl.BlockSpec((BLK, 128), lambda i: (i, 0)),
                  pl.BlockSpec((128,), lambda i: (0,)),
                  pl.BlockSpec((128, PK * C), lambda i: (0, 0)),
                  pl.BlockSpec((PK * C,), lambda i: (0,)),
                  pl.BlockSpec((128, 128), lambda i: (0, 0))],
        out_specs=pl.BlockSpec((BLK, PK * C), lambda i: (i, 0)),
        out_shape=jax.ShapeDtypeStruct((NPK, PK * C), jnp.float32),
    )(acc2v, cntv, r2p, jnp.tile(b2l, PK), bd(Wlin), jnp.tile(blin, PK), bd1)
    return out.reshape(NPAD, C)[:N]
